# baseline (speedup 1.0000x reference)
import functools

import jax
import jax.numpy as jnp
from jax import lax
from jax.experimental import pallas as pl
from jax.experimental.pallas import tpu as pltpu
from jax.experimental.pallas import tpu_sc as plsc

NUM_CORES = 2
NUM_SUBCORES = 16
NUM_WORKERS = NUM_CORES * NUM_SUBCORES
CHUNK = 2048


@functools.partial(jax.jit, static_argnames=("n", "d"))
def _emb_lookup(idx_flat, table, *, n, d):
    per_w = n // NUM_WORKERS
    steps = per_w // CHUNK
    mesh = plsc.VectorSubcoreMesh(
        core_axis_name="c", subcore_axis_name="s", num_cores=NUM_CORES
    )

    @functools.partial(
        pl.kernel,
        mesh=mesh,
        out_type=jax.ShapeDtypeStruct((n, d), jnp.float32),
        compiler_params=pltpu.CompilerParams(use_tc_tiling_on_sc=False),
        scratch_types=[
            pltpu.VMEM((CHUNK,), jnp.int32),
            pltpu.VMEM((CHUNK, d), jnp.float32),
            pltpu.SemaphoreType.DMA,
        ],
    )
    def emb(idx_hbm, tab_hbm, out_hbm, idx_v, rows_v, sem):
        wid = lax.axis_index("s") * NUM_CORES + lax.axis_index("c")
        base = wid * per_w

        def body(i, _):
            off = base + i * CHUNK
            pltpu.sync_copy(idx_hbm.at[pl.ds(off, CHUNK)], idx_v)
            pltpu.async_copy(tab_hbm.at[idx_v], rows_v, sem).wait()
            pltpu.sync_copy(rows_v, out_hbm.at[pl.ds(off, CHUNK)])
            return ()

        lax.fori_loop(0, steps, body, ())

    return emb(idx_flat, table)


def kernel(indices, table):
    b, f = indices.shape
    v, d = table.shape
    n = b * f
    out = _emb_lookup(indices.reshape(n), table, n=n, d=d)
    return out.reshape(b, f, d)


# speedup vs baseline: 1.0053x; 1.0053x over previous
import functools

import jax
import jax.numpy as jnp
from jax import lax
from jax.experimental import pallas as pl
from jax.experimental.pallas import tpu as pltpu
from jax.experimental.pallas import tpu_sc as plsc

NUM_CORES = 2
NUM_SUBCORES = 16
NUM_WORKERS = NUM_CORES * NUM_SUBCORES
CHUNK = 2048


@functools.partial(jax.jit, static_argnames=("n", "d"))
def _emb_lookup(idx_flat, table, *, n, d):
    per_w = n // NUM_WORKERS
    steps = per_w // CHUNK
    mesh = plsc.VectorSubcoreMesh(
        core_axis_name="c", subcore_axis_name="s", num_cores=NUM_CORES
    )

    @functools.partial(
        pl.kernel,
        mesh=mesh,
        out_type=jax.ShapeDtypeStruct((n, d), jnp.float32),
        compiler_params=pltpu.CompilerParams(use_tc_tiling_on_sc=False),
        scratch_types=[
            pltpu.VMEM((per_w,), jnp.int32),
            pltpu.VMEM((2, CHUNK, d), jnp.float32),
            pltpu.SemaphoreType.DMA,
            pltpu.SemaphoreType.DMA((2,)),
            pltpu.SemaphoreType.DMA((2,)),
        ],
    )
    def emb(idx_hbm, tab_hbm, out_hbm, idx_v, rows_v, sem_idx, sem_gat, sem_out):
        wid = lax.axis_index("s") * NUM_CORES + lax.axis_index("c")
        base = wid * per_w

        pltpu.async_copy(idx_hbm.at[pl.ds(base, per_w)], idx_v, sem_idx).wait()

        def gather(i, slot):
            return pltpu.async_copy(
                tab_hbm.at[idx_v.at[pl.ds(i * CHUNK, CHUNK)]],
                rows_v.at[slot],
                sem_gat.at[slot],
            )

        def writeback(i, slot):
            return pltpu.async_copy(
                rows_v.at[slot],
                out_hbm.at[pl.ds(base + i * CHUNK, CHUNK)],
                sem_out.at[slot],
            )

        outs = [None, None]
        g = gather(0, 0)
        for i in range(steps):
            slot = i % 2
            g.wait()
            if i + 1 < steps:
                if outs[1 - slot] is not None:
                    outs[1 - slot].wait()
                g = gather(i + 1, 1 - slot)
            outs[slot] = writeback(i, slot)
        for o in outs:
            if o is not None:
                o.wait()

    return emb(idx_flat, table)


def kernel(indices, table):
    b, f = indices.shape
    v, d = table.shape
    n = b * f
    out = _emb_lookup(indices.reshape(n), table, n=n, d=d)
    return out.reshape(b, f, d)


# speedup vs baseline: 2.4927x; 2.4794x over previous
import functools

import jax
import jax.numpy as jnp
from jax import lax
from jax.experimental import pallas as pl
from jax.experimental.pallas import tpu as pltpu
from jax.experimental.pallas import tpu_sc as plsc

NUM_CORES = 2
NUM_SUBCORES = 16
NUM_WORKERS = NUM_CORES * NUM_SUBCORES
BATCH_PER_CHUNK = 32


@functools.partial(jax.jit, static_argnames=("b", "f", "d"))
def _emb_lookup(idx_flat, table, *, b, f, d):
    batch_per_w = b // NUM_WORKERS
    steps = batch_per_w // BATCH_PER_CHUNK
    chunk = BATCH_PER_CHUNK * f
    mesh = plsc.VectorSubcoreMesh(
        core_axis_name="c", subcore_axis_name="s", num_cores=NUM_CORES
    )

    @functools.partial(
        pl.kernel,
        mesh=mesh,
        out_type=jax.ShapeDtypeStruct((b, f, d), jnp.float32),
        compiler_params=pltpu.CompilerParams(use_tc_tiling_on_sc=False),
        scratch_types=[
            pltpu.VMEM((2, chunk), jnp.int32),
            pltpu.VMEM((2, chunk, d), jnp.float32),
            pltpu.SemaphoreType.DMA((2,)),
            pltpu.SemaphoreType.DMA((2,)),
            pltpu.SemaphoreType.DMA((2,)),
        ],
    )
    def emb(idx_hbm, tab_hbm, out_hbm, idx_v, rows_v, sem_idx, sem_gat, sem_out):
        wid = lax.axis_index("s") * NUM_CORES + lax.axis_index("c")
        b0 = wid * batch_per_w

        def load_idx(i, s):
            return pltpu.async_copy(
                idx_hbm.at[pl.ds((b0 + i * BATCH_PER_CHUNK) * f, chunk)],
                idx_v.at[s],
                sem_idx.at[s],
            )

        idx_cp = [None, None]
        out_cps = [[], []]
        idx_cp[0] = load_idx(0, 0)
        for i in range(steps):
            s = i % 2
            if i + 1 < steps:
                idx_cp[1 - s] = load_idx(i + 1, 1 - s)
            idx_cp[s].wait()
            for cp in out_cps[s]:
                cp.wait()
            out_cps[s] = []
            pltpu.async_copy(
                tab_hbm.at[idx_v.at[s]], rows_v.at[s], sem_gat.at[s]
            ).wait()
            for k in range(BATCH_PER_CHUNK):
                out_cps[s].append(
                    pltpu.async_copy(
                        rows_v.at[s].at[pl.ds(k * f, f)],
                        out_hbm.at[b0 + i * BATCH_PER_CHUNK + k],
                        sem_out.at[s],
                    )
                )
        for cps in out_cps:
            for cp in cps:
                cp.wait()

    return emb(idx_flat, table)


def kernel(indices, table):
    b, f = indices.shape
    v, d = table.shape
    return _emb_lookup(indices.reshape(b * f), table, b=b, f=f, d=d)
